# baseline (speedup 1.0000x reference)
import functools
import numpy as np
import jax
import jax.numpy as jnp
from jax import lax
from jax.experimental import pallas as pl
from jax.experimental.pallas import tpu as pltpu
from jax.experimental.pallas import tpu_sc as plsc

NS = 4
B = 64
V = 100000
W = 8192
NV = (V + W - 1) // W
RB = 8
NB = B // RB
ROWS = NS * RB
C = 512
NCHUNK = W // C

NVTC = 9
VC = NVTC * W
VS = V - VC
NVS = NV - NVTC
RTOT = NS * B

_KS = (np.uint32(0), np.uint32(42), np.uint32(0 ^ 42 ^ 0x1BD11BDA))
_ROT = ((13, 15, 26, 6), (17, 29, 16, 24))


def _rotl(x, r):
    return lax.shift_left(x, np.uint32(r)) | lax.shift_right_logical(
        x, np.uint32(32 - r))


def _tf_bits(x1):
    x0 = jnp.zeros(x1.shape, jnp.uint32)
    for gr in range(5):
        for r in _ROT[gr % 2]:
            x0 = x0 + x1
            x1 = _rotl(x1, r)
            x1 = x1 ^ x0
        x0 = x0 + _KS[(gr + 1) % 3]
        x1 = x1 + (_KS[(gr + 2) % 3] + np.uint32(gr + 1))
    return x0 ^ x1


def _gumbel_from_bits(bits):
    fb = lax.shift_right_logical(bits, np.uint32(9)) | np.uint32(0x3F800000)
    floats = lax.bitcast_convert_type(fb, jnp.float32) - np.float32(1.0)
    u = floats + np.float32(1e-20)
    return -jnp.log(-jnp.log(u))


_SC_UNROLL = 2
_SC_ITERS = VS // (16 * _SC_UNROLL)
_SC_TAIL = (VS % (16 * _SC_UNROLL)) // 16


def _sc_bits_body(bits_hbm, buf, sem_a, sem_b):
    w = lax.axis_index("s") * 2 + lax.axis_index("c")
    iota16 = lax.iota(jnp.uint32, 16)
    sems = (sem_a, sem_b)
    copies = [None, None]
    for k in range(8):
        slot = k & 1
        if copies[slot] is not None:
            copies[slot].wait()
        row = w * 8 + k
        base = (row * V + (VC + 42)).astype(jnp.uint32)

        def body(i, carry, _slot=slot, _base=base):
            off = i * (16 * _SC_UNROLL)
            for t in range(_SC_UNROLL):
                o = off + t * 16
                x1 = iota16 + (_base + o.astype(jnp.uint32))
                buf[_slot, pl.ds(o, 16)] = _tf_bits(x1)
            return carry

        lax.fori_loop(0, _SC_ITERS, body, 0)
        for t in range(_SC_TAIL):
            o = _SC_ITERS * (16 * _SC_UNROLL) + t * 16
            x1 = iota16 + (base + np.uint32(o))
            buf[slot, pl.ds(o, 16)] = _tf_bits(x1)
        copies[slot] = pltpu.async_copy(buf.at[slot], bits_hbm.at[row],
                                        sems[slot])
    copies[0].wait()
    copies[1].wait()


@functools.cache
def _make_sc_bits():
    return pl.kernel(
        _sc_bits_body,
        out_type=jax.ShapeDtypeStruct((RTOT, VS), jnp.uint32),
        mesh=plsc.VectorSubcoreMesh(core_axis_name="c", subcore_axis_name="s"),
        scratch_types=[
            pltpu.VMEM((2, VS), jnp.uint32),
            pltpu.SemaphoreType.DMA,
            pltpu.SemaphoreType.DMA,
        ],
    )


def _argmax_kernel(probs_ref, sumo_ref, max_ref, idx_ref, sums_s, max_s,
                   idx_s):
    ph = pl.program_id(0)
    bi = pl.program_id(1)
    vi = pl.program_id(2)

    @pl.when(ph == 0)
    def _():
        col = vi * W + lax.broadcasted_iota(jnp.int32, (RB, W), 1)
        x = jnp.where(col < V, probs_ref[...], np.float32(0.0))
        part = jnp.broadcast_to(jnp.sum(x, axis=-1, keepdims=True), (RB, 128))
        acc = jnp.where(vi == 0, part, sums_s[bi] + part)
        sums_s[bi] = acc

        @pl.when(vi == NV - 1)
        def _():
            sumo_ref[...] = acc[:, 0:1]

    @pl.when((ph == 1) & (vi < NVTC))
    def _():
        @pl.when(vi == 0)
        def _():
            max_s[...] = jnp.full((ROWS, 128), -jnp.inf, jnp.float32)
            idx_s[...] = jnp.zeros((ROWS, 128), jnp.int32)

        p = probs_ref[...] / sums_s[bi][:, 0:1]
        logp = jnp.log(p)

        s3 = lax.broadcasted_iota(jnp.int32, (NS, RB, 1), 0)
        lb3 = lax.broadcasted_iota(jnp.int32, (NS, RB, 1), 1)
        base3 = s3 * (B * V) + (bi * RB + lb3) * V + (vi * W + 42)
        lane3 = lax.broadcasted_iota(jnp.int32, (NS, RB, C), 2)

        runmax = jnp.full((NS, RB, C), -jnp.inf, jnp.float32)
        runchunk = jnp.zeros((NS, RB, C), jnp.int32)
        for c in range(NCHUNK):
            x1 = ((base3 + c * C) + lane3).astype(jnp.uint32)
            g = _gumbel_from_bits(_tf_bits(x1))
            score = logp[None, :, c * C:(c + 1) * C] + g
            better = score > runmax
            runmax = jnp.where(better, score, runmax)
            runchunk = jnp.where(better, jnp.int32(c), runchunk)

        bm = jnp.max(runmax, axis=-1, keepdims=True)
        eq = runmax == bm
        cand = jnp.where(eq, runchunk * C + lane3, jnp.int32(2**31 - 1))
        bidx = vi * W + jnp.min(cand, axis=-1, keepdims=True)

        bm2 = bm.reshape(ROWS, 1)
        bidx2 = bidx.reshape(ROWS, 1)
        rm = max_s[:, 0:1]
        ri = idx_s[:, 0:1]
        better2 = bm2 > rm
        nm = jnp.where(better2, bm2, rm)
        ni = jnp.where(better2, bidx2, ri)
        max_s[...] = jnp.broadcast_to(nm, (ROWS, 128))
        idx_s[...] = jnp.broadcast_to(ni, (ROWS, 128))

        @pl.when(vi == NVTC - 1)
        def _():
            max_ref[...] = jnp.broadcast_to(nm, (ROWS, 128)).reshape(
                1, ROWS, 128)
            idx_ref[...] = jnp.broadcast_to(ni, (ROWS, 128)).reshape(
                1, ROWS, 128)


def _merge_kernel(probs_ref, sum_ref, bits_ref, am_ref, ai_ref, out_ref,
                  max_s, idx_s):
    vj = pl.program_id(1)

    @pl.when(vj == 0)
    def _():
        max_s[...] = am_ref[0]
        idx_s[...] = ai_ref[0]

    @pl.when(vj < NVS)
    def _():
        p = probs_ref[...] / sum_ref[...]
        logp = jnp.log(p)
        lane3 = lax.broadcasted_iota(jnp.int32, (NS, RB, C), 2)

        runmax = jnp.full((NS, RB, C), -jnp.inf, jnp.float32)
        runchunk = jnp.zeros((NS, RB, C), jnp.int32)
        for c in range(NCHUNK):
            bits = bits_ref[:, :, c * C:(c + 1) * C]
            g = _gumbel_from_bits(bits)
            score = logp[None, :, c * C:(c + 1) * C] + g
            thresh = V - VC - vj * W - c * C
            score = jnp.where(lane3 < thresh, score, -jnp.inf)
            better = score > runmax
            runmax = jnp.where(better, score, runmax)
            runchunk = jnp.where(better, jnp.int32(c), runchunk)

        bm = jnp.max(runmax, axis=-1, keepdims=True)
        eq = runmax == bm
        cand = jnp.where(eq, runchunk * C + lane3, jnp.int32(2**31 - 1))
        bidx = (VC + vj * W) + jnp.min(cand, axis=-1, keepdims=True)

        bm2 = bm.reshape(ROWS, 1)
        bidx2 = bidx.reshape(ROWS, 1)
        rm = max_s[:, 0:1]
        ri = idx_s[:, 0:1]
        better2 = bm2 > rm
        nm = jnp.where(better2, bm2, rm)
        ni = jnp.where(better2, bidx2, ri)
        max_s[...] = jnp.broadcast_to(nm, (ROWS, 128))
        idx_s[...] = jnp.broadcast_to(ni, (ROWS, 128))

    @pl.when(vj >= NVS)
    def _():
        ids = idx_s[:, 0:1]
        col = (vj - NVS) * W + lax.broadcasted_iota(jnp.int32, (ROWS, W), 1)
        oh = jnp.where(col == ids, np.float32(1.0), np.float32(0.0))
        out_ref[...] = oh.reshape(NS, RB, W)


@jax.jit
def kernel(probs):
    bits = _make_sc_bits()()
    bits3 = bits.reshape(NS, B, VS)

    sums, am, ai = pl.pallas_call(
        _argmax_kernel,
        grid=(2, NB, NV),
        in_specs=[
            pl.BlockSpec((RB, W), lambda ph, bi, vi: (bi, vi)),
        ],
        out_specs=[
            pl.BlockSpec((RB, 1),
                         lambda ph, bi, vi: (jnp.where(ph == 0, bi, NB), 0)),
            pl.BlockSpec((1, ROWS, 128), lambda ph, bi, vi: (bi, 0, 0)),
            pl.BlockSpec((1, ROWS, 128), lambda ph, bi, vi: (bi, 0, 0)),
        ],
        out_shape=[
            jax.ShapeDtypeStruct(((NB + 1) * RB, 1), jnp.float32),
            jax.ShapeDtypeStruct((NB, ROWS, 128), jnp.float32),
            jax.ShapeDtypeStruct((NB, ROWS, 128), jnp.int32),
        ],
        scratch_shapes=[
            pltpu.VMEM((NB, RB, 128), jnp.float32),
            pltpu.VMEM((ROWS, 128), jnp.float32),
            pltpu.VMEM((ROWS, 128), jnp.int32),
        ],
    )(probs)

    out = pl.pallas_call(
        _merge_kernel,
        grid=(NB, NVS + NV),
        in_specs=[
            pl.BlockSpec((RB, W),
                         lambda bi, vj: (bi, jnp.minimum(NVTC + vj, NV - 1))),
            pl.BlockSpec((RB, 1), lambda bi, vj: (bi, 0)),
            pl.BlockSpec((NS, RB, W),
                         lambda bi, vj: (0, bi, jnp.minimum(vj, NVS - 1))),
            pl.BlockSpec((1, ROWS, 128), lambda bi, vj: (bi, 0, 0)),
            pl.BlockSpec((1, ROWS, 128), lambda bi, vj: (bi, 0, 0)),
        ],
        out_specs=pl.BlockSpec(
            (NS, RB, W),
            lambda bi, vj: (0, bi, jnp.maximum(vj - NVS, 0))),
        out_shape=jax.ShapeDtypeStruct((NS, B, V), jnp.float32),
        scratch_shapes=[
            pltpu.VMEM((ROWS, 128), jnp.float32),
            pltpu.VMEM((ROWS, 128), jnp.int32),
        ],
    )(probs, sums, bits3, am, ai)
    return out


# speedup vs baseline: 1.1358x; 1.1358x over previous
import functools
import numpy as np
import jax
import jax.numpy as jnp
from jax import lax
from jax.experimental import pallas as pl
from jax.experimental.pallas import tpu as pltpu
from jax.experimental.pallas import tpu_sc as plsc

NS = 4
B = 64
V = 100000
W = 8192
NV = (V + W - 1) // W
RB = 8
NB = B // RB
ROWS = NS * RB
C = 512
NCHUNK = W // C

NVTC = 9
VC = NVTC * W
VS = V - VC
NVS = NV - NVTC
RTOT = NS * B

_KS = (np.uint32(0), np.uint32(42), np.uint32(0 ^ 42 ^ 0x1BD11BDA))
_ROT = ((13, 15, 26, 6), (17, 29, 16, 24))


def _rotl(x, r):
    return lax.shift_left(x, np.uint32(r)) | lax.shift_right_logical(
        x, np.uint32(32 - r))


def _tf_bits(x1):
    x0 = jnp.zeros(x1.shape, jnp.uint32)
    for gr in range(5):
        for r in _ROT[gr % 2]:
            x0 = x0 + x1
            x1 = _rotl(x1, r)
            x1 = x1 ^ x0
        x0 = x0 + _KS[(gr + 1) % 3]
        x1 = x1 + (_KS[(gr + 2) % 3] + np.uint32(gr + 1))
    return x0 ^ x1


def _gumbel_from_bits(bits):
    fb = lax.shift_right_logical(bits, np.uint32(9)) | np.uint32(0x3F800000)
    floats = lax.bitcast_convert_type(fb, jnp.float32) - np.float32(1.0)
    u = floats + np.float32(1e-20)
    return -jnp.log(-jnp.log(u))


_SC_UNROLL = 2
_SC_ITERS = VS // (16 * _SC_UNROLL)
_SC_TAIL = (VS % (16 * _SC_UNROLL)) // 16


def _sc_bits_body(bits_hbm, buf, sem_a, sem_b):
    w = lax.axis_index("s") * 2 + lax.axis_index("c")
    iota16 = lax.iota(jnp.uint32, 16)
    sems = (sem_a, sem_b)
    copies = [None, None]
    for k in range(8):
        slot = k & 1
        if copies[slot] is not None:
            copies[slot].wait()
        row = w * 8 + k
        base = (row * V + (VC + 42)).astype(jnp.uint32)

        def body(i, carry, _slot=slot, _base=base):
            off = i * (16 * _SC_UNROLL)
            for t in range(_SC_UNROLL):
                o = off + t * 16
                x1 = iota16 + (_base + o.astype(jnp.uint32))
                buf[_slot, pl.ds(o, 16)] = _tf_bits(x1)
            return carry

        lax.fori_loop(0, _SC_ITERS, body, 0)
        for t in range(_SC_TAIL):
            o = _SC_ITERS * (16 * _SC_UNROLL) + t * 16
            x1 = iota16 + (base + np.uint32(o))
            buf[slot, pl.ds(o, 16)] = _tf_bits(x1)
        copies[slot] = pltpu.async_copy(buf.at[slot], bits_hbm.at[row],
                                        sems[slot])
    copies[0].wait()
    copies[1].wait()


@functools.cache
def _make_sc_bits():
    return pl.kernel(
        _sc_bits_body,
        out_type=jax.ShapeDtypeStruct((RTOT, VS), jnp.uint32),
        mesh=plsc.VectorSubcoreMesh(core_axis_name="c", subcore_axis_name="s"),
        scratch_types=[
            pltpu.VMEM((2, VS), jnp.uint32),
            pltpu.SemaphoreType.DMA,
            pltpu.SemaphoreType.DMA,
        ],
    )


def _sum_kernel(probs_ref, sum_ref):
    vi = pl.program_id(0)
    col = vi * W + lax.broadcasted_iota(jnp.int32, (B, W), 1)
    x = jnp.where(col < V, probs_ref[...], np.float32(0.0))
    part = jnp.sum(x, axis=-1, keepdims=True)

    @pl.when(vi == 0)
    def _():
        sum_ref[...] = part

    @pl.when(vi > 0)
    def _():
        sum_ref[...] = sum_ref[...] + part


def _argmax_kernel(probs_ref, sum_ref, max_ref, idx_ref, max_s, idx_s):
    bi = pl.program_id(0)
    vi = pl.program_id(1)

    @pl.when(vi == 0)
    def _():
        max_s[...] = jnp.full((ROWS, 128), -jnp.inf, jnp.float32)
        idx_s[...] = jnp.zeros((ROWS, 128), jnp.int32)

    p = probs_ref[...] / sum_ref[...]
    logp = jnp.log(p)

    s3 = lax.broadcasted_iota(jnp.int32, (NS, RB, 1), 0)
    lb3 = lax.broadcasted_iota(jnp.int32, (NS, RB, 1), 1)
    base3 = s3 * (B * V) + (bi * RB + lb3) * V + (vi * W + 42)
    lane3 = lax.broadcasted_iota(jnp.int32, (NS, RB, C), 2)

    runmax = jnp.full((NS, RB, C), -jnp.inf, jnp.float32)
    runchunk = jnp.zeros((NS, RB, C), jnp.int32)
    for c in range(NCHUNK):
        x1 = ((base3 + c * C) + lane3).astype(jnp.uint32)
        g = _gumbel_from_bits(_tf_bits(x1))
        score = logp[None, :, c * C:(c + 1) * C] + g
        better = score > runmax
        runmax = jnp.where(better, score, runmax)
        runchunk = jnp.where(better, jnp.int32(c), runchunk)

    bm = jnp.max(runmax, axis=-1, keepdims=True)
    eq = runmax == bm
    cand = jnp.where(eq, runchunk * C + lane3, jnp.int32(2**31 - 1))
    bidx = vi * W + jnp.min(cand, axis=-1, keepdims=True)

    bm2 = bm.reshape(ROWS, 1)
    bidx2 = bidx.reshape(ROWS, 1)
    rm = max_s[:, 0:1]
    ri = idx_s[:, 0:1]
    better2 = bm2 > rm
    nm = jnp.where(better2, bm2, rm)
    ni = jnp.where(better2, bidx2, ri)
    max_s[...] = jnp.broadcast_to(nm, (ROWS, 128))
    idx_s[...] = jnp.broadcast_to(ni, (ROWS, 128))

    @pl.when(vi == NVTC - 1)
    def _():
        max_ref[...] = jnp.broadcast_to(nm, (ROWS, 128)).reshape(1, ROWS, 128)
        idx_ref[...] = jnp.broadcast_to(ni, (ROWS, 128)).reshape(1, ROWS, 128)


def _merge_kernel(probs_ref, sum_ref, bits_ref, am_ref, ai_ref, out_ref,
                  max_s, idx_s):
    vj = pl.program_id(1)

    @pl.when(vj == 0)
    def _():
        max_s[...] = am_ref[0]
        idx_s[...] = ai_ref[0]

    @pl.when(vj < NVS)
    def _():
        p = probs_ref[...] / sum_ref[...]
        logp = jnp.log(p)
        lane3 = lax.broadcasted_iota(jnp.int32, (NS, RB, C), 2)

        runmax = jnp.full((NS, RB, C), -jnp.inf, jnp.float32)
        runchunk = jnp.zeros((NS, RB, C), jnp.int32)
        for c in range(NCHUNK):
            bits = bits_ref[:, :, c * C:(c + 1) * C]
            g = _gumbel_from_bits(bits)
            score = logp[None, :, c * C:(c + 1) * C] + g
            thresh = V - VC - vj * W - c * C
            score = jnp.where(lane3 < thresh, score, -jnp.inf)
            better = score > runmax
            runmax = jnp.where(better, score, runmax)
            runchunk = jnp.where(better, jnp.int32(c), runchunk)

        bm = jnp.max(runmax, axis=-1, keepdims=True)
        eq = runmax == bm
        cand = jnp.where(eq, runchunk * C + lane3, jnp.int32(2**31 - 1))
        bidx = (VC + vj * W) + jnp.min(cand, axis=-1, keepdims=True)

        bm2 = bm.reshape(ROWS, 1)
        bidx2 = bidx.reshape(ROWS, 1)
        rm = max_s[:, 0:1]
        ri = idx_s[:, 0:1]
        better2 = bm2 > rm
        nm = jnp.where(better2, bm2, rm)
        ni = jnp.where(better2, bidx2, ri)
        max_s[...] = jnp.broadcast_to(nm, (ROWS, 128))
        idx_s[...] = jnp.broadcast_to(ni, (ROWS, 128))

    @pl.when(vj >= NVS)
    def _():
        ids = idx_s[:, 0:1]
        col = (vj - NVS) * W + lax.broadcasted_iota(jnp.int32, (ROWS, W), 1)
        oh = jnp.where(col == ids, np.float32(1.0), np.float32(0.0))
        out_ref[...] = oh.reshape(NS, RB, W)


@jax.jit
def kernel(probs):
    sums = pl.pallas_call(
        _sum_kernel,
        grid=(NV,),
        in_specs=[pl.BlockSpec((B, W), lambda vi: (0, vi))],
        out_specs=pl.BlockSpec((B, 1), lambda vi: (0, 0)),
        out_shape=jax.ShapeDtypeStruct((B, 1), jnp.float32),
    )(probs)

    bits = _make_sc_bits()()
    bits3 = bits.reshape(NS, B, VS)

    am, ai = pl.pallas_call(
        _argmax_kernel,
        grid=(NB, NVTC),
        in_specs=[
            pl.BlockSpec((RB, W), lambda bi, vi: (bi, vi)),
            pl.BlockSpec((RB, 1), lambda bi, vi: (bi, 0)),
        ],
        out_specs=[
            pl.BlockSpec((1, ROWS, 128), lambda bi, vi: (bi, 0, 0)),
            pl.BlockSpec((1, ROWS, 128), lambda bi, vi: (bi, 0, 0)),
        ],
        out_shape=[
            jax.ShapeDtypeStruct((NB, ROWS, 128), jnp.float32),
            jax.ShapeDtypeStruct((NB, ROWS, 128), jnp.int32),
        ],
        scratch_shapes=[
            pltpu.VMEM((ROWS, 128), jnp.float32),
            pltpu.VMEM((ROWS, 128), jnp.int32),
        ],
    )(probs, sums)

    out = pl.pallas_call(
        _merge_kernel,
        grid=(NB, NVS + NV),
        in_specs=[
            pl.BlockSpec((RB, W),
                         lambda bi, vj: (bi, jnp.minimum(NVTC + vj, NV - 1))),
            pl.BlockSpec((RB, 1), lambda bi, vj: (bi, 0)),
            pl.BlockSpec((NS, RB, W),
                         lambda bi, vj: (0, bi, jnp.minimum(vj, NVS - 1))),
            pl.BlockSpec((1, ROWS, 128), lambda bi, vj: (bi, 0, 0)),
            pl.BlockSpec((1, ROWS, 128), lambda bi, vj: (bi, 0, 0)),
        ],
        out_specs=pl.BlockSpec(
            (NS, RB, W),
            lambda bi, vj: (0, bi, jnp.maximum(vj - NVS, 0))),
        out_shape=jax.ShapeDtypeStruct((NS, B, V), jnp.float32),
        scratch_shapes=[
            pltpu.VMEM((ROWS, 128), jnp.float32),
            pltpu.VMEM((ROWS, 128), jnp.int32),
        ],
    )(probs, sums, bits3, am, ai)
    return out


# speedup vs baseline: 1.1574x; 1.0190x over previous
import functools
import numpy as np
import jax
import jax.numpy as jnp
from jax import lax
from jax.experimental import pallas as pl
from jax.experimental.pallas import tpu as pltpu
from jax.experimental.pallas import tpu_sc as plsc

NS = 4
B = 64
V = 100000
W = 8192
NV = (V + W - 1) // W
RB = 8
NB = B // RB
ROWS = NS * RB
C = 512
NCHUNK = W // C

NVTC = 9
VC = NVTC * W
VS = V - VC
NVS = NV - NVTC
RTOT = NS * B

_KS = (np.uint32(0), np.uint32(42), np.uint32(0 ^ 42 ^ 0x1BD11BDA))
_ROT = ((13, 15, 26, 6), (17, 29, 16, 24))


def _rotl(x, r):
    return lax.shift_left(x, np.uint32(r)) | lax.shift_right_logical(
        x, np.uint32(32 - r))


def _tf_bits(x1):
    x0 = jnp.zeros(x1.shape, jnp.uint32)
    for gr in range(5):
        for r in _ROT[gr % 2]:
            x0 = x0 + x1
            x1 = _rotl(x1, r)
            x1 = x1 ^ x0
        x0 = x0 + _KS[(gr + 1) % 3]
        x1 = x1 + (_KS[(gr + 2) % 3] + np.uint32(gr + 1))
    return x0 ^ x1


def _gumbel_from_bits(bits):
    fb = lax.shift_right_logical(bits, np.uint32(9)) | np.uint32(0x3F800000)
    floats = lax.bitcast_convert_type(fb, jnp.float32) - np.float32(1.0)
    u = floats + np.float32(1e-20)
    return -jnp.log(-jnp.log(u))


_SC_UNROLL = 2
_SC_ITERS = VS // (16 * _SC_UNROLL)
_SC_TAIL = (VS % (16 * _SC_UNROLL)) // 16


def _sc_bits_body(bits_hbm, buf, sem_a, sem_b):
    w = lax.axis_index("s") * 2 + lax.axis_index("c")
    iota16 = lax.iota(jnp.uint32, 16)
    sems = (sem_a, sem_b)
    copies = [None, None]
    for k in range(8):
        slot = k & 1
        if copies[slot] is not None:
            copies[slot].wait()
        row = w * 8 + k
        base = (row * V + (VC + 42)).astype(jnp.uint32)

        def body(i, carry, _slot=slot, _base=base):
            off = i * (16 * _SC_UNROLL)
            for t in range(_SC_UNROLL):
                o = off + t * 16
                x1 = iota16 + (_base + o.astype(jnp.uint32))
                buf[_slot, pl.ds(o, 16)] = _tf_bits(x1)
            return carry

        lax.fori_loop(0, _SC_ITERS, body, 0)
        for t in range(_SC_TAIL):
            o = _SC_ITERS * (16 * _SC_UNROLL) + t * 16
            x1 = iota16 + (base + np.uint32(o))
            buf[slot, pl.ds(o, 16)] = _tf_bits(x1)
        copies[slot] = pltpu.async_copy(buf.at[slot], bits_hbm.at[row],
                                        sems[slot])
    copies[0].wait()
    copies[1].wait()


@functools.cache
def _make_sc_bits():
    return pl.kernel(
        _sc_bits_body,
        out_type=jax.ShapeDtypeStruct((RTOT, VS), jnp.uint32),
        mesh=plsc.VectorSubcoreMesh(core_axis_name="c", subcore_axis_name="s"),
        scratch_types=[
            pltpu.VMEM((2, VS), jnp.uint32),
            pltpu.SemaphoreType.DMA,
            pltpu.SemaphoreType.DMA,
        ],
    )


def _sum_kernel(probs_ref, sum_ref):
    vi = pl.program_id(0)
    col = vi * W + lax.broadcasted_iota(jnp.int32, (B, W), 1)
    x = jnp.where(col < V, probs_ref[...], np.float32(0.0))
    part = jnp.sum(x, axis=-1, keepdims=True)

    @pl.when(vi == 0)
    def _():
        sum_ref[...] = part

    @pl.when(vi > 0)
    def _():
        sum_ref[...] = sum_ref[...] + part


def _argmax_kernel(probs_ref, sum_ref, max_ref, idx_ref, rmax_s, rpos_s):
    bi = pl.program_id(0)
    vi = pl.program_id(1)

    @pl.when(vi == 0)
    def _():
        rmax_s[...] = jnp.full((NS, RB, C), -jnp.inf, jnp.float32)
        rpos_s[...] = jnp.zeros((NS, RB, C), jnp.int32)

    p = probs_ref[...] / sum_ref[...]
    logp = jnp.log(p)

    s3 = lax.broadcasted_iota(jnp.int32, (NS, RB, 1), 0)
    lb3 = lax.broadcasted_iota(jnp.int32, (NS, RB, 1), 1)
    base3 = s3 * (B * V) + (bi * RB + lb3) * V + (vi * W + 42)
    lane3 = lax.broadcasted_iota(jnp.int32, (NS, RB, C), 2)

    runmax = rmax_s[...]
    runpos = rpos_s[...]
    for c in range(NCHUNK):
        x1 = ((base3 + c * C) + lane3).astype(jnp.uint32)
        g = _gumbel_from_bits(_tf_bits(x1))
        score = logp[None, :, c * C:(c + 1) * C] + g
        better = score > runmax
        runmax = jnp.where(better, score, runmax)
        runpos = jnp.where(better, jnp.int32(vi * NCHUNK + c), runpos)
    rmax_s[...] = runmax
    rpos_s[...] = runpos

    @pl.when(vi == NVTC - 1)
    def _():
        bm = jnp.max(runmax, axis=-1, keepdims=True)
        eq = runmax == bm
        cand = jnp.where(eq, runpos * C + lane3, jnp.int32(2**31 - 1))
        bidx = jnp.min(cand, axis=-1, keepdims=True)
        bm2 = jnp.broadcast_to(bm.reshape(ROWS, 1), (ROWS, 128))
        bi2 = jnp.broadcast_to(bidx.reshape(ROWS, 1), (ROWS, 128))
        max_ref[...] = bm2.reshape(1, ROWS, 128)
        idx_ref[...] = bi2.reshape(1, ROWS, 128)


def _merge_kernel(probs_ref, sum_ref, bits_ref, am_ref, ai_ref, out_ref,
                  rmax_s, rpos_s, idxf_s):
    vj = pl.program_id(1)

    @pl.when(vj == 0)
    def _():
        rmax_s[...] = jnp.full((NS, RB, C), -jnp.inf, jnp.float32)
        rpos_s[...] = jnp.zeros((NS, RB, C), jnp.int32)

    @pl.when(vj < NVS)
    def _():
        p = probs_ref[...] / sum_ref[...]
        logp = jnp.log(p)
        lane3 = lax.broadcasted_iota(jnp.int32, (NS, RB, C), 2)

        runmax = rmax_s[...]
        runpos = rpos_s[...]
        for c in range(NCHUNK):
            bits = bits_ref[:, :, c * C:(c + 1) * C]
            g = _gumbel_from_bits(bits)
            score = logp[None, :, c * C:(c + 1) * C] + g
            thresh = V - VC - vj * W - c * C
            score = jnp.where(lane3 < thresh, score, -jnp.inf)
            better = score > runmax
            runmax = jnp.where(better, score, runmax)
            runpos = jnp.where(better, jnp.int32(vj * NCHUNK + c), runpos)
        rmax_s[...] = runmax
        rpos_s[...] = runpos

        @pl.when(vj == NVS - 1)
        def _():
            bm = jnp.max(runmax, axis=-1, keepdims=True)
            eq = runmax == bm
            cand = jnp.where(eq, runpos * C + lane3, jnp.int32(2**31 - 1))
            bidx = VC + jnp.min(cand, axis=-1, keepdims=True)

            bm2 = bm.reshape(ROWS, 1)
            bidx2 = bidx.reshape(ROWS, 1)
            rm = am_ref[0, :, 0:1]
            ri = ai_ref[0, :, 0:1]
            better2 = bm2 > rm
            ni = jnp.where(better2, bidx2, ri)
            idxf_s[...] = jnp.broadcast_to(ni, (ROWS, 128))

    @pl.when(vj >= NVS)
    def _():
        ids = idxf_s[:, 0:1]
        col = (vj - NVS) * W + lax.broadcasted_iota(jnp.int32, (ROWS, W), 1)
        oh = jnp.where(col == ids, np.float32(1.0), np.float32(0.0))
        out_ref[...] = oh.reshape(NS, RB, W)


@jax.jit
def kernel(probs):
    sums = pl.pallas_call(
        _sum_kernel,
        grid=(NV,),
        in_specs=[pl.BlockSpec((B, W), lambda vi: (0, vi))],
        out_specs=pl.BlockSpec((B, 1), lambda vi: (0, 0)),
        out_shape=jax.ShapeDtypeStruct((B, 1), jnp.float32),
    )(probs)

    bits = _make_sc_bits()()
    bits3 = bits.reshape(NS, B, VS)

    am, ai = pl.pallas_call(
        _argmax_kernel,
        grid=(NB, NVTC),
        in_specs=[
            pl.BlockSpec((RB, W), lambda bi, vi: (bi, vi)),
            pl.BlockSpec((RB, 1), lambda bi, vi: (bi, 0)),
        ],
        out_specs=[
            pl.BlockSpec((1, ROWS, 128), lambda bi, vi: (bi, 0, 0)),
            pl.BlockSpec((1, ROWS, 128), lambda bi, vi: (bi, 0, 0)),
        ],
        out_shape=[
            jax.ShapeDtypeStruct((NB, ROWS, 128), jnp.float32),
            jax.ShapeDtypeStruct((NB, ROWS, 128), jnp.int32),
        ],
        scratch_shapes=[
            pltpu.VMEM((NS, RB, C), jnp.float32),
            pltpu.VMEM((NS, RB, C), jnp.int32),
        ],
    )(probs, sums)

    out = pl.pallas_call(
        _merge_kernel,
        grid=(NB, NVS + NV),
        in_specs=[
            pl.BlockSpec((RB, W),
                         lambda bi, vj: (bi, jnp.minimum(NVTC + vj, NV - 1))),
            pl.BlockSpec((RB, 1), lambda bi, vj: (bi, 0)),
            pl.BlockSpec((NS, RB, W),
                         lambda bi, vj: (0, bi, jnp.minimum(vj, NVS - 1))),
            pl.BlockSpec((1, ROWS, 128), lambda bi, vj: (bi, 0, 0)),
            pl.BlockSpec((1, ROWS, 128), lambda bi, vj: (bi, 0, 0)),
        ],
        out_specs=pl.BlockSpec(
            (NS, RB, W),
            lambda bi, vj: (0, bi, jnp.maximum(vj - NVS, 0))),
        out_shape=jax.ShapeDtypeStruct((NS, B, V), jnp.float32),
        scratch_shapes=[
            pltpu.VMEM((NS, RB, C), jnp.float32),
            pltpu.VMEM((NS, RB, C), jnp.int32),
            pltpu.VMEM((ROWS, 128), jnp.int32),
        ],
    )(probs, sums, bits3, am, ai)
    return out


# speedup vs baseline: 1.1666x; 1.0080x over previous
import functools
import numpy as np
import jax
import jax.numpy as jnp
from jax import lax
from jax.experimental import pallas as pl
from jax.experimental.pallas import tpu as pltpu
from jax.experimental.pallas import tpu_sc as plsc

NS = 4
B = 64
V = 100000
W = 8192
NV = (V + W - 1) // W
RB = 8
NB = B // RB
ROWS = NS * RB
C = 512
NCHUNK = W // C

NVTC = 9
VC = NVTC * W
VS = V - VC
NVS = NV - NVTC
RTOT = NS * B

_KS = (np.uint32(0), np.uint32(42), np.uint32(0 ^ 42 ^ 0x1BD11BDA))
_ROT = ((13, 15, 26, 6), (17, 29, 16, 24))


def _rotl(x, r):
    return lax.shift_left(x, np.uint32(r)) | lax.shift_right_logical(
        x, np.uint32(32 - r))


def _tf_bits(x1):
    x0 = jnp.zeros(x1.shape, jnp.uint32)
    for gr in range(5):
        for r in _ROT[gr % 2]:
            x0 = x0 + x1
            x1 = _rotl(x1, r)
            x1 = x1 ^ x0
        x0 = x0 + _KS[(gr + 1) % 3]
        x1 = x1 + (_KS[(gr + 2) % 3] + np.uint32(gr + 1))
    return x0 ^ x1


def _gumbel_from_bits(bits):
    fb = lax.shift_right_logical(bits, np.uint32(9)) | np.uint32(0x3F800000)
    floats = lax.bitcast_convert_type(fb, jnp.float32) - np.float32(1.0)
    return -jnp.log(-jnp.log(floats))


_SC_UNROLL = 2
_SC_ITERS = VS // (16 * _SC_UNROLL)
_SC_TAIL = (VS % (16 * _SC_UNROLL)) // 16


def _sc_bits_body(bits_hbm, buf, sem_a, sem_b):
    w = lax.axis_index("s") * 2 + lax.axis_index("c")
    iota16 = lax.iota(jnp.uint32, 16)
    sems = (sem_a, sem_b)
    copies = [None, None]
    for k in range(8):
        slot = k & 1
        if copies[slot] is not None:
            copies[slot].wait()
        row = w * 8 + k
        base = (row * V + (VC + 42)).astype(jnp.uint32)

        def body(i, carry, _slot=slot, _base=base):
            off = i * (16 * _SC_UNROLL)
            for t in range(_SC_UNROLL):
                o = off + t * 16
                x1 = iota16 + (_base + o.astype(jnp.uint32))
                buf[_slot, pl.ds(o, 16)] = _tf_bits(x1)
            return carry

        lax.fori_loop(0, _SC_ITERS, body, 0)
        for t in range(_SC_TAIL):
            o = _SC_ITERS * (16 * _SC_UNROLL) + t * 16
            x1 = iota16 + (base + np.uint32(o))
            buf[slot, pl.ds(o, 16)] = _tf_bits(x1)
        copies[slot] = pltpu.async_copy(buf.at[slot], bits_hbm.at[row],
                                        sems[slot])
    copies[0].wait()
    copies[1].wait()


@functools.cache
def _make_sc_bits():
    return pl.kernel(
        _sc_bits_body,
        out_type=jax.ShapeDtypeStruct((RTOT, VS), jnp.uint32),
        mesh=plsc.VectorSubcoreMesh(core_axis_name="c", subcore_axis_name="s"),
        scratch_types=[
            pltpu.VMEM((2, VS), jnp.uint32),
            pltpu.SemaphoreType.DMA,
            pltpu.SemaphoreType.DMA,
        ],
    )


def _sum_kernel(probs_ref, sum_ref):
    vi = pl.program_id(0)
    col = vi * W + lax.broadcasted_iota(jnp.int32, (B, W), 1)
    x = jnp.where(col < V, probs_ref[...], np.float32(0.0))
    part = jnp.sum(x, axis=-1, keepdims=True)

    @pl.when(vi == 0)
    def _():
        sum_ref[...] = part

    @pl.when(vi > 0)
    def _():
        sum_ref[...] = sum_ref[...] + part


def _argmax_kernel(probs_ref, sum_ref, max_ref, idx_ref, rmax_s, rpos_s):
    bi = pl.program_id(0)
    vi = pl.program_id(1)

    @pl.when(vi == 0)
    def _():
        rmax_s[...] = jnp.full((NS, RB, C), -jnp.inf, jnp.float32)
        rpos_s[...] = jnp.zeros((NS, RB, C), jnp.int32)

    p = probs_ref[...] / sum_ref[...]
    logp = jnp.log(p)

    s3 = lax.broadcasted_iota(jnp.int32, (NS, RB, 1), 0)
    lb3 = lax.broadcasted_iota(jnp.int32, (NS, RB, 1), 1)
    base3 = s3 * (B * V) + (bi * RB + lb3) * V + (vi * W + 42)
    lane3 = lax.broadcasted_iota(jnp.int32, (NS, RB, C), 2)

    runmax = rmax_s[...]
    runpos = rpos_s[...]
    for c in range(NCHUNK):
        x1 = ((base3 + c * C) + lane3).astype(jnp.uint32)
        g = _gumbel_from_bits(_tf_bits(x1))
        score = logp[None, :, c * C:(c + 1) * C] + g
        better = score > runmax
        runmax = jnp.where(better, score, runmax)
        runpos = jnp.where(better, jnp.int32(vi * NCHUNK + c), runpos)
    rmax_s[...] = runmax
    rpos_s[...] = runpos

    @pl.when(vi == NVTC - 1)
    def _():
        bm = jnp.max(runmax, axis=-1, keepdims=True)
        eq = runmax == bm
        cand = jnp.where(eq, runpos * C + lane3, jnp.int32(2**31 - 1))
        bidx = jnp.min(cand, axis=-1, keepdims=True)
        bm2 = jnp.broadcast_to(bm.reshape(ROWS, 1), (ROWS, 128))
        bi2 = jnp.broadcast_to(bidx.reshape(ROWS, 1), (ROWS, 128))
        max_ref[...] = bm2.reshape(1, ROWS, 128)
        idx_ref[...] = bi2.reshape(1, ROWS, 128)


def _merge_kernel(probs_ref, sum_ref, bits_ref, am_ref, ai_ref, out_ref,
                  rmax_s, rpos_s, idxf_s):
    vj = pl.program_id(1)

    @pl.when(vj == 0)
    def _():
        rmax_s[...] = jnp.full((NS, RB, C), -jnp.inf, jnp.float32)
        rpos_s[...] = jnp.zeros((NS, RB, C), jnp.int32)

    @pl.when(vj < NVS)
    def _():
        p = probs_ref[...] / sum_ref[...]
        logp = jnp.log(p)
        lane3 = lax.broadcasted_iota(jnp.int32, (NS, RB, C), 2)

        runmax = rmax_s[...]
        runpos = rpos_s[...]
        for c in range(NCHUNK):
            bits = bits_ref[:, :, c * C:(c + 1) * C]
            g = _gumbel_from_bits(bits)
            score = logp[None, :, c * C:(c + 1) * C] + g
            thresh = V - VC - vj * W - c * C
            score = jnp.where(lane3 < thresh, score, -jnp.inf)
            better = score > runmax
            runmax = jnp.where(better, score, runmax)
            runpos = jnp.where(better, jnp.int32(vj * NCHUNK + c), runpos)
        rmax_s[...] = runmax
        rpos_s[...] = runpos

        @pl.when(vj == NVS - 1)
        def _():
            bm = jnp.max(runmax, axis=-1, keepdims=True)
            eq = runmax == bm
            cand = jnp.where(eq, runpos * C + lane3, jnp.int32(2**31 - 1))
            bidx = VC + jnp.min(cand, axis=-1, keepdims=True)

            bm2 = bm.reshape(ROWS, 1)
            bidx2 = bidx.reshape(ROWS, 1)
            rm = am_ref[0, :, 0:1]
            ri = ai_ref[0, :, 0:1]
            better2 = bm2 > rm
            ni = jnp.where(better2, bidx2, ri)
            idxf_s[...] = jnp.broadcast_to(ni, (ROWS, 128))

    @pl.when(vj >= NVS)
    def _():
        ids = idxf_s[:, 0:1]
        col = (vj - NVS) * W + lax.broadcasted_iota(jnp.int32, (ROWS, W), 1)
        oh = jnp.where(col == ids, np.float32(1.0), np.float32(0.0))
        out_ref[...] = oh.reshape(NS, RB, W)


@jax.jit
def kernel(probs):
    sums = pl.pallas_call(
        _sum_kernel,
        grid=(NV,),
        in_specs=[pl.BlockSpec((B, W), lambda vi: (0, vi))],
        out_specs=pl.BlockSpec((B, 1), lambda vi: (0, 0)),
        out_shape=jax.ShapeDtypeStruct((B, 1), jnp.float32),
    )(probs)

    bits = _make_sc_bits()()
    bits3 = bits.reshape(NS, B, VS)

    am, ai = pl.pallas_call(
        _argmax_kernel,
        grid=(NB, NVTC),
        in_specs=[
            pl.BlockSpec((RB, W), lambda bi, vi: (bi, vi)),
            pl.BlockSpec((RB, 1), lambda bi, vi: (bi, 0)),
        ],
        out_specs=[
            pl.BlockSpec((1, ROWS, 128), lambda bi, vi: (bi, 0, 0)),
            pl.BlockSpec((1, ROWS, 128), lambda bi, vi: (bi, 0, 0)),
        ],
        out_shape=[
            jax.ShapeDtypeStruct((NB, ROWS, 128), jnp.float32),
            jax.ShapeDtypeStruct((NB, ROWS, 128), jnp.int32),
        ],
        scratch_shapes=[
            pltpu.VMEM((NS, RB, C), jnp.float32),
            pltpu.VMEM((NS, RB, C), jnp.int32),
        ],
    )(probs, sums)

    out = pl.pallas_call(
        _merge_kernel,
        grid=(NB, NVS + NV),
        in_specs=[
            pl.BlockSpec((RB, W),
                         lambda bi, vj: (bi, jnp.minimum(NVTC + vj, NV - 1))),
            pl.BlockSpec((RB, 1), lambda bi, vj: (bi, 0)),
            pl.BlockSpec((NS, RB, W),
                         lambda bi, vj: (0, bi, jnp.minimum(vj, NVS - 1))),
            pl.BlockSpec((1, ROWS, 128), lambda bi, vj: (bi, 0, 0)),
            pl.BlockSpec((1, ROWS, 128), lambda bi, vj: (bi, 0, 0)),
        ],
        out_specs=pl.BlockSpec(
            (NS, RB, W),
            lambda bi, vj: (0, bi, jnp.maximum(vj - NVS, 0))),
        out_shape=jax.ShapeDtypeStruct((NS, B, V), jnp.float32),
        scratch_shapes=[
            pltpu.VMEM((NS, RB, C), jnp.float32),
            pltpu.VMEM((NS, RB, C), jnp.int32),
            pltpu.VMEM((ROWS, 128), jnp.int32),
        ],
    )(probs, sums, bits3, am, ai)
    return out
